# Initial kernel scaffold; baseline (speedup 1.0000x reference)
import jax
import jax.numpy as jnp
from jax.experimental import pallas as pl


def kernel(inputs, support, W, b):
    raise NotImplementedError("write your pallas kernel here")


# speedup vs baseline: 2.7588x; 2.7588x over previous
import functools

import jax
import jax.numpy as jnp
from jax.experimental import pallas as pl
from jax.experimental.pallas import tpu as pltpu

N = 1024
F = 128
OUT = 128
K = 9
BI = 8
CHUNK = 8
NEG = float("-inf")


def _body(adj_ref, fea_ref, w_ref, b_ref, out_ref, topk_ref):
    for d in range(BI):

        def s1_body(c, R, d=d):
            fea_c = fea_ref[pl.ds(c * CHUNK, CHUNK), :]
            a_c = adj_ref[pl.ds(c * CHUNK, CHUNK), 0, 0, d : d + 1]
            v = fea_c * a_c
            out = []
            for t in range(K):
                hi = jnp.maximum(R[t], v)
                v = jnp.minimum(R[t], v)
                out.append(hi)
            return tuple(out)

        R0 = tuple(jnp.full((CHUNK, F), NEG) for _ in range(K))
        R = jax.lax.fori_loop(0, N // CHUNK, s1_body, R0, unroll=2)
        cand = jnp.concatenate(R, axis=0)

        def s2_body(t, m):
            topk_ref[pl.ds(t * BI + d, 1), :] = m
            return jnp.max(jnp.where(cand < m, cand, NEG), axis=0,
                           keepdims=True)

        m0 = jnp.max(cand, axis=0, keepdims=True)
        jax.lax.fori_loop(0, K, s2_body, m0)

    acc = jnp.zeros((BI, OUT), dtype=jnp.float32)
    for t in range(K):
        acc += jnp.dot(topk_ref[t * BI : (t + 1) * BI, :], w_ref[t],
                       preferred_element_type=jnp.float32)
    out_ref[:] = jnp.maximum(acc + b_ref[:], 0.0)


@jax.jit
def kernel(inputs, support, W, b):
    b2 = b.reshape(1, OUT)
    support4 = support.reshape(N, N // BI, 1, BI)
    grid = (N // BI,)
    return pl.pallas_call(
        _body,
        grid=grid,
        in_specs=[
            pl.BlockSpec((N, 1, 1, BI), lambda ib: (0, ib, 0, 0)),
            pl.BlockSpec((N, F), lambda ib: (0, 0)),
            pl.BlockSpec((K, F, OUT), lambda ib: (0, 0, 0)),
            pl.BlockSpec((1, OUT), lambda ib: (0, 0)),
        ],
        out_specs=pl.BlockSpec((BI, OUT), lambda ib: (ib, 0)),
        out_shape=jax.ShapeDtypeStruct((N, OUT), jnp.float32),
        scratch_shapes=[pltpu.VMEM((K * BI, F), jnp.float32)],
    )(support4, inputs, W, b2)


# speedup vs baseline: 10.3656x; 3.7573x over previous
import jax
import jax.numpy as jnp
from jax.experimental import pallas as pl
from jax.experimental.pallas import tpu as pltpu

N = 1024
F = 128
OUT = 128
K = 9
BI = 8
CHUNK = 16
NEGB = float(-3e38)


def _body(adj_ref, fea_ref, w_ref, b_ref, out_ref, topk_ref, p_ref):
    for d in range(BI):
        a_col = adj_ref[:, 0, 0, d : d + 1]
        p_ref[:] = fea_ref[:] * a_col

        def s1_body(c, R):
            v = p_ref[pl.ds(c * CHUNK, CHUNK), :]
            out = []
            for t in range(K):
                hi = jnp.maximum(R[t], v)
                v = jnp.minimum(R[t], v)
                out.append(hi)
            return tuple(out)

        R0 = tuple(jnp.full((CHUNK, F), NEGB, dtype=jnp.bfloat16)
                   for _ in range(K))
        R = jax.lax.fori_loop(0, N // CHUNK, s1_body, R0, unroll=8)
        cand = jnp.concatenate([r.astype(jnp.float32) for r in R], axis=0)
        ids = jax.lax.broadcasted_iota(jnp.int32, (K * CHUNK, F), 0)
        keys = jax.lax.bitcast_convert_type(
            jax.lax.bitcast_convert_type(cand, jnp.int32) | ids, jnp.float32)

        def s2_body(t, m, d=d):
            val = jax.lax.bitcast_convert_type(
                jax.lax.bitcast_convert_type(m, jnp.int32) & (~0xFF),
                jnp.float32)
            topk_ref[pl.ds(t * BI + d, 1), :] = val
            return jnp.max(jnp.where(keys < m, keys, NEGB), axis=0,
                           keepdims=True)

        m0 = jnp.max(keys, axis=0, keepdims=True)
        jax.lax.fori_loop(0, K, s2_body, m0)

    acc = jnp.zeros((BI, OUT), dtype=jnp.float32)
    for t in range(K):
        acc += jnp.dot(topk_ref[t * BI : (t + 1) * BI, :], w_ref[t],
                       preferred_element_type=jnp.float32)
    out_ref[:] = jnp.maximum(acc + b_ref[:], 0.0)


@jax.jit
def kernel(inputs, support, W, b):
    b2 = b.reshape(1, OUT)
    fea = inputs.astype(jnp.bfloat16)
    support4 = support.astype(jnp.bfloat16).reshape(N, N // BI, 1, BI)
    grid = (N // BI,)
    return pl.pallas_call(
        _body,
        grid=grid,
        in_specs=[
            pl.BlockSpec((N, 1, 1, BI), lambda ib: (0, ib, 0, 0)),
            pl.BlockSpec((N, F), lambda ib: (0, 0)),
            pl.BlockSpec((K, F, OUT), lambda ib: (0, 0, 0)),
            pl.BlockSpec((1, OUT), lambda ib: (0, 0)),
        ],
        out_specs=pl.BlockSpec((BI, OUT), lambda ib: (ib, 0)),
        out_shape=jax.ShapeDtypeStruct((N, OUT), jnp.float32),
        scratch_shapes=[
            pltpu.VMEM((K * BI, F), jnp.float32),
            pltpu.VMEM((N, F), jnp.bfloat16),
        ],
    )(support4, fea, W, b2)


# speedup vs baseline: 28.1966x; 2.7202x over previous
import jax
import jax.numpy as jnp
from jax.experimental import pallas as pl
from jax.experimental.pallas import tpu as pltpu

N = 1024
F = 128
OUT = 128
K = 9
BI = 8
CHUNK = 16
NEGB = float(-3e38)


def _body(adj_ref, fea_ref, w_ref, b_ref, out_ref, topk_ref, pan_ref):
    a8t = adj_ref[0]
    dsel = (jax.lax.broadcasted_iota(jnp.int32, (BI, BI * F), 0)
            == jax.lax.broadcasted_iota(jnp.int32, (BI, BI * F), 1) // F
            ).astype(jnp.bfloat16)
    pan_ref[:] = jax.lax.dot_general(
        a8t, dsel, (((0,), (0,)), ((), ())),
        preferred_element_type=jnp.float32).astype(jnp.bfloat16)
    for d in range(BI):

        def s1_body(c, R, d=d):
            v = (pan_ref[pl.ds(c * CHUNK, CHUNK), d * F : (d + 1) * F]
                 * fea_ref[pl.ds(c * CHUNK, CHUNK), :])
            out = []
            for t in range(K):
                hi = jnp.maximum(R[t], v)
                v = jnp.minimum(R[t], v)
                out.append(hi)
            return tuple(out)

        R0 = tuple(jnp.full((CHUNK, F), NEGB, dtype=jnp.bfloat16)
                   for _ in range(K))
        R = jax.lax.fori_loop(0, N // CHUNK, s1_body, R0, unroll=8)
        cand = jnp.concatenate([r.astype(jnp.float32) for r in R], axis=0)
        ids = jax.lax.broadcasted_iota(jnp.int32, (K * CHUNK, F), 0)
        keys = jax.lax.bitcast_convert_type(
            jax.lax.bitcast_convert_type(cand, jnp.int32) | ids, jnp.float32)

        def s2_body(t, m, d=d):
            val = jax.lax.bitcast_convert_type(
                jax.lax.bitcast_convert_type(m, jnp.int32) & (~0xFF),
                jnp.float32)
            topk_ref[pl.ds(t * BI + d, 1), :] = val
            return jnp.max(jnp.where(keys < m, keys, NEGB), axis=0,
                           keepdims=True)

        m0 = jnp.max(keys, axis=0, keepdims=True)
        jax.lax.fori_loop(0, K, s2_body, m0)

    acc = jnp.zeros((BI, OUT), dtype=jnp.float32)
    for t in range(K):
        acc += jnp.dot(topk_ref[t * BI : (t + 1) * BI, :], w_ref[t],
                       preferred_element_type=jnp.float32)
    out_ref[:] = jnp.maximum(acc + b_ref[:], 0.0)


@jax.jit
def kernel(inputs, support, W, b):
    b2 = b.reshape(1, OUT)
    fea = inputs.astype(jnp.bfloat16)
    support3 = support.T.astype(jnp.bfloat16).reshape(N // BI, BI, N)
    grid = (N // BI,)
    return pl.pallas_call(
        _body,
        grid=grid,
        in_specs=[
            pl.BlockSpec((1, BI, N), lambda ib: (ib, 0, 0)),
            pl.BlockSpec((N, F), lambda ib: (0, 0)),
            pl.BlockSpec((K, F, OUT), lambda ib: (0, 0, 0)),
            pl.BlockSpec((1, OUT), lambda ib: (0, 0)),
        ],
        out_specs=pl.BlockSpec((BI, OUT), lambda ib: (ib, 0)),
        out_shape=jax.ShapeDtypeStruct((N, OUT), jnp.float32),
        scratch_shapes=[
            pltpu.VMEM((K * BI, F), jnp.float32),
            pltpu.VMEM((N, BI * F), jnp.bfloat16),
        ],
    )(support3, fea, W, b2)


# speedup vs baseline: 29.8581x; 1.0589x over previous
import jax
import jax.numpy as jnp
from jax.experimental import pallas as pl
from jax.experimental.pallas import tpu as pltpu

N = 1024
F = 128
OUT = 128
K = 9
BI = 8
CHUNK = 16
NEGB = float(-3e38)


def _body(adj_ref, fea_ref, w_ref, b_ref, out_ref, topk_ref, pan_ref):
    a8t = adj_ref[0]
    dsel = (jax.lax.broadcasted_iota(jnp.int32, (BI, BI * F), 0)
            == jax.lax.broadcasted_iota(jnp.int32, (BI, BI * F), 1) // F
            ).astype(jnp.bfloat16)
    pan_ref[:] = jax.lax.dot_general(
        a8t, dsel, (((0,), (0,)), ((), ())),
        preferred_element_type=jnp.float32).astype(jnp.bfloat16)
    for d in range(BI):

        def s1_body(c, R, d=d):
            v = (pan_ref[pl.ds(c * CHUNK, CHUNK), d * F : (d + 1) * F]
                 * fea_ref[pl.ds(c * CHUNK, CHUNK), :])
            out = []
            for t in range(K):
                hi = jnp.maximum(R[t], v)
                v = jnp.minimum(R[t], v)
                out.append(hi)
            return tuple(out)

        R0 = tuple(jnp.full((CHUNK, F), NEGB, dtype=jnp.bfloat16)
                   for _ in range(K))
        R = jax.lax.fori_loop(0, N // CHUNK, s1_body, R0, unroll=8)
        c32 = jnp.concatenate([r.astype(jnp.float32) for r in R], axis=0)
        cand = jnp.concatenate(
            [jnp.maximum(c32[t * CHUNK : t * CHUNK + 8, :],
                         c32[(8 - t) * CHUNK + 8 : (8 - t) * CHUNK + 16, :])
             for t in range(K)], axis=0)
        ids = jax.lax.broadcasted_iota(jnp.int32, (K * 8, F), 0)
        keys = jax.lax.bitcast_convert_type(
            jax.lax.bitcast_convert_type(cand, jnp.int32) | ids, jnp.float32)

        def s2_body(t, m, d=d):
            val = jax.lax.bitcast_convert_type(
                jax.lax.bitcast_convert_type(m, jnp.int32) & (~0xFF),
                jnp.float32)
            topk_ref[pl.ds(t * BI + d, 1), :] = val
            return jnp.max(jnp.where(keys < m, keys, NEGB), axis=0,
                           keepdims=True)

        m0 = jnp.max(keys, axis=0, keepdims=True)
        jax.lax.fori_loop(0, K, s2_body, m0)

    acc = jnp.zeros((BI, OUT), dtype=jnp.float32)
    for t in range(K):
        acc += jnp.dot(topk_ref[t * BI : (t + 1) * BI, :], w_ref[t],
                       preferred_element_type=jnp.float32)
    out_ref[:] = jnp.maximum(acc + b_ref[:], 0.0)


@jax.jit
def kernel(inputs, support, W, b):
    b2 = b.reshape(1, OUT)
    fea = inputs.astype(jnp.bfloat16)
    support3 = support.T.astype(jnp.bfloat16).reshape(N // BI, BI, N)
    grid = (N // BI,)
    return pl.pallas_call(
        _body,
        grid=grid,
        in_specs=[
            pl.BlockSpec((1, BI, N), lambda ib: (ib, 0, 0)),
            pl.BlockSpec((N, F), lambda ib: (0, 0)),
            pl.BlockSpec((K, F, OUT), lambda ib: (0, 0, 0)),
            pl.BlockSpec((1, OUT), lambda ib: (0, 0)),
        ],
        out_specs=pl.BlockSpec((BI, OUT), lambda ib: (ib, 0)),
        out_shape=jax.ShapeDtypeStruct((N, OUT), jnp.float32),
        scratch_shapes=[
            pltpu.VMEM((K * BI, F), jnp.float32),
            pltpu.VMEM((N, BI * F), jnp.bfloat16),
        ],
    )(support3, fea, W, b2)


# speedup vs baseline: 31.6955x; 1.0615x over previous
import jax
import jax.numpy as jnp
from jax.experimental import pallas as pl
from jax.experimental.pallas import tpu as pltpu

N = 1024
F = 128
OUT = 128
K = 9
BI = 8
CHUNK = 16
NEGB = float(-3e38)


def _body(adj_ref, fea_ref, dsel_ref, w_ref, b_ref, out_ref, topk_ref,
          pan_ref):
    a8t = adj_ref[0]
    pan_ref[:] = jax.lax.dot_general(
        a8t, dsel_ref[:], (((0,), (0,)), ((), ())),
        preferred_element_type=jnp.float32).astype(jnp.bfloat16)
    for d in range(BI):

        def s1_body(c, R, d=d):
            v = (pan_ref[pl.ds(c * CHUNK, CHUNK), d * F : (d + 1) * F]
                 * fea_ref[pl.ds(c * CHUNK, CHUNK), :])
            out = []
            for t in range(K):
                hi = jnp.maximum(R[t], v)
                v = jnp.minimum(R[t], v)
                out.append(hi)
            return tuple(out)

        R0 = tuple(jnp.full((CHUNK, F), NEGB, dtype=jnp.bfloat16)
                   for _ in range(K))
        R = jax.lax.fori_loop(0, N // CHUNK, s1_body, R0, unroll=16)
        c32 = jnp.concatenate([r.astype(jnp.float32) for r in R], axis=0)
        cand = jnp.concatenate(
            [jnp.maximum(c32[t * CHUNK : t * CHUNK + 8, :],
                         c32[(8 - t) * CHUNK + 8 : (8 - t) * CHUNK + 16, :])
             for t in range(K)], axis=0)
        ids = jax.lax.broadcasted_iota(jnp.int32, (K * 8, F), 0)
        keys = jax.lax.bitcast_convert_type(
            jax.lax.bitcast_convert_type(cand, jnp.int32) | ids, jnp.float32)

        def s2_body(t, m, d=d):
            val = jax.lax.bitcast_convert_type(
                jax.lax.bitcast_convert_type(m, jnp.int32) & (~0xFF),
                jnp.float32)
            topk_ref[pl.ds(t * BI + d, 1), :] = val
            return jnp.max(jnp.where(keys < m, keys, NEGB), axis=0,
                           keepdims=True)

        m0 = jnp.max(keys, axis=0, keepdims=True)
        jax.lax.fori_loop(0, K, s2_body, m0)

    acc = jnp.zeros((BI, OUT), dtype=jnp.float32)
    for t in range(K):
        acc += jnp.dot(topk_ref[t * BI : (t + 1) * BI, :], w_ref[t],
                       preferred_element_type=jnp.float32)
    out_ref[:] = jnp.maximum(acc + b_ref[:], 0.0)


@jax.jit
def kernel(inputs, support, W, b):
    b2 = b.reshape(1, OUT)
    fea = inputs.astype(jnp.bfloat16)
    support3 = support.T.astype(jnp.bfloat16).reshape(N // BI, BI, N)
    dsel = (jnp.arange(BI, dtype=jnp.int32)[:, None]
            == (jnp.arange(BI * F, dtype=jnp.int32) // F)[None, :]
            ).astype(jnp.bfloat16)
    grid = (N // BI,)
    return pl.pallas_call(
        _body,
        grid=grid,
        in_specs=[
            pl.BlockSpec((1, BI, N), lambda ib: (ib, 0, 0)),
            pl.BlockSpec((N, F), lambda ib: (0, 0)),
            pl.BlockSpec((BI, BI * F), lambda ib: (0, 0)),
            pl.BlockSpec((K, F, OUT), lambda ib: (0, 0, 0)),
            pl.BlockSpec((1, OUT), lambda ib: (0, 0)),
        ],
        out_specs=pl.BlockSpec((BI, OUT), lambda ib: (ib, 0)),
        out_shape=jax.ShapeDtypeStruct((N, OUT), jnp.float32),
        scratch_shapes=[
            pltpu.VMEM((K * BI, F), jnp.float32),
            pltpu.VMEM((N, BI * F), jnp.bfloat16),
        ],
    )(support3, fea, dsel, W, b2)
